# baseline (speedup 1.0000x reference)
import jax
import jax.numpy as jnp
from jax import lax
from jax.experimental import pallas as pl
from jax.experimental.pallas import tpu as pltpu
from jax.experimental.pallas import tpu_sc as plsc

_B = 16384
_D = 64
_NC = 2
_NS = 16
_NW = _NC * _NS
_PER_W = _B // _NW
_SUB = _PER_W // 128
_G = _PER_W // 16


def _body(s_hbm, r_hbm, o_hbm, et_hbm, rt_hbm, out_hbm,
          s_ix, r_ix, o_ix, s_bf, r_bf, o_bf, out_v, sem):
    cid = lax.axis_index("c")
    sid = lax.axis_index("s")
    wid = sid * _NC + cid
    base = wid * _PER_W

    for hbm, ix in ((s_hbm, s_ix), (r_hbm, r_ix), (o_hbm, o_ix)):
        for k in range(_SUB):
            pltpu.sync_copy(hbm.at[pl.ds(base + k * 128, 128)], ix.at[k, 0])

    copies = []
    for j in range(_D):
        for tab, ix, bf in ((et_hbm, s_ix, s_bf), (rt_hbm, r_ix, r_bf),
                            (et_hbm, o_ix, o_bf)):
            for k in range(_SUB):
                copies.append(pltpu.async_copy(
                    tab.at[pl.ds(j, 1)].at[ix.at[k]], bf.at[j, k], sem))
    for cp in copies:
        cp.wait()

    for g in range(_G):
        sub, off = g // 8, (g % 8) * 16

        def j_body(j, acc, sub=sub, off=off):
            lane = pl.ds(off, 16)
            sv = s_bf[j, sub, 0, lane]
            rv = r_bf[j, sub, 0, lane]
            ov = o_bf[j, sub, 0, lane]
            return acc + jnp.abs(sv + rv - ov)

        acc = lax.fori_loop(0, _D, j_body, jnp.zeros((16,), jnp.float32))
        out_v[pl.ds(g * 16, 16)] = -acc

    pltpu.sync_copy(out_v, out_hbm.at[pl.ds(base, _PER_W)])


@jax.jit
def _transe_sc(s, r, o, et, rt):
    mesh = plsc.VectorSubcoreMesh(core_axis_name="c", subcore_axis_name="s")
    return pl.kernel(
        _body,
        mesh=mesh,
        compiler_params=pltpu.CompilerParams(
            needs_layout_passes=False, use_tc_tiling_on_sc=False),
        out_type=jax.ShapeDtypeStruct((_B,), jnp.float32),
        scratch_types=[
            pltpu.VMEM((_SUB, 1, 128), jnp.int32),
            pltpu.VMEM((_SUB, 1, 128), jnp.int32),
            pltpu.VMEM((_SUB, 1, 128), jnp.int32),
            pltpu.VMEM((_D, _SUB, 1, 128), jnp.float32),
            pltpu.VMEM((_D, _SUB, 1, 128), jnp.float32),
            pltpu.VMEM((_D, _SUB, 1, 128), jnp.float32),
            pltpu.VMEM((_PER_W,), jnp.float32),
            pltpu.SemaphoreType.DMA,
        ],
    )(s, r, o, et, rt)


def kernel(s, r, o, E_center, R_center):
    return _transe_sc(s, r, o, E_center.T, R_center.T)


# speedup vs baseline: 7.6214x; 7.6214x over previous
import jax
import jax.numpy as jnp
from jax import lax
from jax.experimental import pallas as pl
from jax.experimental.pallas import tpu as pltpu
from jax.experimental.pallas import tpu_sc as plsc

_B = 16384
_D = 64
_NC = 2
_NS = 16
_NW = _NC * _NS
_PER_W = _B // _NW
_HP = _PER_W // 2
_UNROLL = 8


def _body(s_hbm, r_hbm, o_hbm, e_hbm, rel_hbm, out_hbm,
          idx_sh, si_v, ri_v, oi_v, s_rows, r_rows, o_rows, out_v,
          s_sm, r_sm, o_sm, sem):
    cid = lax.axis_index("c")
    sid = lax.axis_index("s")
    wid = sid * _NC + cid
    base = wid * _PER_W

    for hbm, vm, sm in ((s_hbm, si_v, s_sm), (r_hbm, ri_v, r_sm),
                        (o_hbm, oi_v, o_sm)):
        pltpu.sync_copy(hbm.at[pl.ds(base, _PER_W)], idx_sh.at[sid])
        pltpu.sync_copy(idx_sh.at[sid], sm)
        pltpu.sync_copy(hbm.at[pl.ds(base, _PER_W)], vm)

    lanes = lax.iota(jnp.int32, 16)

    for p in range(2):
        poff = p * _HP

        def fire(i, _, poff=poff):
            for k in range(_UNROLL):
                row = i * _UNROLL + k
                dst = pl.ds(row, 1)
                pltpu.async_copy(e_hbm.at[pl.ds(s_sm[poff + row] >> 1, 1)],
                                 s_rows.at[dst], sem)
                pltpu.async_copy(rel_hbm.at[pl.ds(r_sm[poff + row] >> 1, 1)],
                                 r_rows.at[dst], sem)
                pltpu.async_copy(e_hbm.at[pl.ds(o_sm[poff + row] >> 1, 1)],
                                 o_rows.at[dst], sem)
            return 0

        lax.fori_loop(0, _HP // _UNROLL, fire, 0)
        pltpu.make_async_copy(e_hbm.at[pl.ds(0, _HP)], s_rows, sem).wait()
        pltpu.make_async_copy(e_hbm.at[pl.ds(0, _HP)], r_rows, sem).wait()
        pltpu.make_async_copy(e_hbm.at[pl.ds(0, _HP)], o_rows, sem).wait()

        for g in range(_HP // 16):
            rows = g * 16 + lanes
            sh = (plsc.load_gather(si_v, [poff + rows]) & 1) << 6
            rh = (plsc.load_gather(ri_v, [poff + rows]) & 1) << 6
            oh = (plsc.load_gather(oi_v, [poff + rows]) & 1) << 6

            def j_body(j, acc, rows=rows, sh=sh, rh=rh, oh=oh):
                sv = plsc.load_gather(s_rows, [rows, sh + j])
                rv = plsc.load_gather(r_rows, [rows, rh + j])
                ov = plsc.load_gather(o_rows, [rows, oh + j])
                return acc + jnp.abs(sv + rv - ov)

            acc = lax.fori_loop(0, _D, j_body, jnp.zeros((16,), jnp.float32))
            out_v[pl.ds(poff + g * 16, 16)] = -acc

    pltpu.sync_copy(out_v, out_hbm.at[pl.ds(base, _PER_W)])


@jax.jit
def _transe_sc(s, r, o, e2, rel2):
    mesh = plsc.VectorSubcoreMesh(core_axis_name="c", subcore_axis_name="s")
    return pl.kernel(
        _body,
        mesh=mesh,
        compiler_params=pltpu.CompilerParams(
            needs_layout_passes=False, use_tc_tiling_on_sc=True),
        out_type=jax.ShapeDtypeStruct((_B,), jnp.float32),
        scratch_types=[
            pltpu.VMEM_SHARED((_NS, _PER_W), jnp.int32),
            pltpu.VMEM((_PER_W,), jnp.int32),
            pltpu.VMEM((_PER_W,), jnp.int32),
            pltpu.VMEM((_PER_W,), jnp.int32),
            pltpu.VMEM((_HP, 128), jnp.float32),
            pltpu.VMEM((_HP, 128), jnp.float32),
            pltpu.VMEM((_HP, 128), jnp.float32),
            pltpu.VMEM((_PER_W,), jnp.float32),
            pltpu.SMEM((_PER_W,), jnp.int32),
            pltpu.SMEM((_PER_W,), jnp.int32),
            pltpu.SMEM((_PER_W,), jnp.int32),
            pltpu.SemaphoreType.DMA,
        ],
    )(s, r, o, e2, rel2)


def kernel(s, r, o, E_center, R_center):
    e2 = E_center.reshape(E_center.shape[0] // 2, 2 * _D)
    rel2 = R_center.reshape(R_center.shape[0] // 2, 2 * _D)
    return _transe_sc(s, r, o, e2, rel2)


# speedup vs baseline: 7.6938x; 1.0095x over previous
import jax
import jax.numpy as jnp
from jax import lax
from jax.experimental import pallas as pl
from jax.experimental.pallas import tpu as pltpu
from jax.experimental.pallas import tpu_sc as plsc

_B = 16384
_D = 64
_NC = 2
_NS = 16
_NW = _NC * _NS
_PER_W = _B // _NW
_SUB = _PER_W // 128
_G = _PER_W // 16


def _body(s_hbm, r_hbm, o_hbm, e_hbm, rel_hbm, out_hbm,
          s_ix, r_ix, o_ix, s_rows, r_rows, o_rows, out_v, sem):
    cid = lax.axis_index("c")
    sid = lax.axis_index("s")
    wid = sid * _NC + cid
    base = wid * _PER_W

    for hbm, ix in ((s_hbm, s_ix), (r_hbm, r_ix), (o_hbm, o_ix)):
        for k in range(_SUB):
            pltpu.sync_copy(hbm.at[pl.ds(base + k * 128, 128)], ix.at[k])

    copies = []
    for k in range(_SUB):
        dst = pl.ds(k * 128, 128)
        copies.append(pltpu.async_copy(e_hbm.at[s_ix.at[k]],
                                       s_rows.at[dst], sem))
        copies.append(pltpu.async_copy(rel_hbm.at[r_ix.at[k]],
                                       r_rows.at[dst], sem))
        copies.append(pltpu.async_copy(e_hbm.at[o_ix.at[k]],
                                       o_rows.at[dst], sem))
    for cp in copies:
        cp.wait()

    lanes = lax.iota(jnp.int32, 16)
    for g in range(_G):
        rows = g * 16 + lanes

        def j_body(j, acc, rows=rows):
            col = jnp.full((16,), 0, jnp.int32) + j
            sv = plsc.load_gather(s_rows, [rows, col])
            rv = plsc.load_gather(r_rows, [rows, col])
            ov = plsc.load_gather(o_rows, [rows, col])
            return acc + jnp.abs(sv + rv - ov)

        acc = lax.fori_loop(0, _D, j_body, jnp.zeros((16,), jnp.float32))
        out_v[pl.ds(g * 16, 16)] = -acc

    pltpu.sync_copy(out_v, out_hbm.at[pl.ds(base, _PER_W)])


@jax.jit
def _transe_sc(s, r, o, e, rel):
    mesh = plsc.VectorSubcoreMesh(core_axis_name="c", subcore_axis_name="s")
    return pl.kernel(
        _body,
        mesh=mesh,
        compiler_params=pltpu.CompilerParams(
            needs_layout_passes=False, use_tc_tiling_on_sc=False),
        out_type=jax.ShapeDtypeStruct((_B,), jnp.float32),
        scratch_types=[
            pltpu.VMEM((_SUB, 128), jnp.int32),
            pltpu.VMEM((_SUB, 128), jnp.int32),
            pltpu.VMEM((_SUB, 128), jnp.int32),
            pltpu.VMEM((_PER_W, _D), jnp.float32),
            pltpu.VMEM((_PER_W, _D), jnp.float32),
            pltpu.VMEM((_PER_W, _D), jnp.float32),
            pltpu.VMEM((_PER_W,), jnp.float32),
            pltpu.SemaphoreType.DMA,
        ],
    )(s, r, o, e, rel)


def kernel(s, r, o, E_center, R_center):
    return _transe_sc(s, r, o, E_center, R_center)
